# Initial kernel scaffold; baseline (speedup 1.0000x reference)
import jax
import jax.numpy as jnp
from jax.experimental import pallas as pl


def kernel(atom_features, pair_features, pair_split, atom_to_pair, W_AA, b_AA, W_PA, b_PA, W_A, b_A, W_AP, b_AP, W_PP, b_PP, W_P, b_P):
    raise NotImplementedError("write your pallas kernel here")


# speedup vs baseline: 2.1949x; 2.1949x over previous
import functools

import jax
import jax.numpy as jnp
from jax import lax
from jax.experimental import pallas as pl
from jax.experimental.pallas import tpu as pltpu
from jax.experimental.pallas import tpu_sc as plsc

F32 = jnp.float32

NC, NS, L = 2, 16, 16
NW = NC * NS


def _prep_body(atom_ref, w1_ref, b1_ref, w2_ref, b2_ref, xcat_ref, aa_ref):
    a = atom_ref[...]
    xcat_ref[...] = jnp.dot(a, w1_ref[...], preferred_element_type=F32) + b1_ref[...]
    aa_ref[...] = jax.nn.relu(
        jnp.dot(a, w2_ref[...], preferred_element_type=F32) + b2_ref[...])


def _tc_prep(atom, w1, b1, w2, b2):
    n, k = atom.shape
    bm = 2000
    grid = (n // bm,)
    return pl.pallas_call(
        _prep_body,
        grid=grid,
        in_specs=[
            pl.BlockSpec((bm, k), lambda i: (i, 0)),
            pl.BlockSpec((k, 128), lambda i: (0, 0)),
            pl.BlockSpec((1, 128), lambda i: (0, 0)),
            pl.BlockSpec((k, 64), lambda i: (0, 0)),
            pl.BlockSpec((1, 64), lambda i: (0, 0)),
        ],
        out_specs=[
            pl.BlockSpec((bm, 128), lambda i: (i, 0)),
            pl.BlockSpec((bm, 64), lambda i: (i, 0)),
        ],
        out_shape=[
            jax.ShapeDtypeStruct((n, 128), F32),
            jax.ShapeDtypeStruct((n, 64), F32),
        ],
    )(atom, w1, b1, w2, b2)


def _pa_body(pair_ref, w_ref, b_ref, out0_ref, out1_ref, out2_ref, out3_ref):
    x = jax.nn.relu(
        jnp.dot(pair_ref[...], w_ref[...], preferred_element_type=F32) + b_ref[...])
    out0_ref[...] = x[:, 0:16]
    out1_ref[...] = x[:, 16:32]
    out2_ref[...] = x[:, 32:48]
    out3_ref[...] = x[:, 48:64]


def _tc_pair_dense(pair, w3, b3):
    n, k = pair.shape
    bm = 8000
    return pl.pallas_call(
        _pa_body,
        grid=(n // bm,),
        in_specs=[
            pl.BlockSpec((bm, k), lambda i: (i, 0)),
            pl.BlockSpec((k, 64), lambda i: (0, 0)),
            pl.BlockSpec((1, 64), lambda i: (0, 0)),
        ],
        out_specs=[pl.BlockSpec((bm, 16), lambda i: (i, 0))] * 4,
        out_shape=[jax.ShapeDtypeStruct((n, 16), F32)] * 4,
    )(pair, w3, b3)


def _p_body(s_ref, pair_ref, wt_ref, wpp_ref, bpp_ref, wb_ref, bp_ref, out_ref):
    pp = jax.nn.relu(
        jnp.dot(pair_ref[...], wpp_ref[...], preferred_element_type=F32)
        + bpp_ref[...])
    acc = (jnp.dot(s_ref[...], wt_ref[...], preferred_element_type=F32)
           + jnp.dot(pp, wb_ref[...], preferred_element_type=F32) + bp_ref[...])
    out_ref[...] = jax.nn.relu(acc)


def _tc_final_p(s64, pair, wt64, wpp, bpp, wpb, bp):
    n = pair.shape[0]
    k = pair.shape[1]
    bm = 8000
    return pl.pallas_call(
        _p_body,
        grid=(n // bm,),
        in_specs=[
            pl.BlockSpec((bm, 64), lambda i: (i, 0)),
            pl.BlockSpec((bm, k), lambda i: (i, 0)),
            pl.BlockSpec((64, 50), lambda i: (0, 0)),
            pl.BlockSpec((k, 50), lambda i: (0, 0)),
            pl.BlockSpec((1, 50), lambda i: (0, 0)),
            pl.BlockSpec((50, 50), lambda i: (0, 0)),
            pl.BlockSpec((1, 50), lambda i: (0, 0)),
        ],
        out_specs=pl.BlockSpec((bm, 50), lambda i: (i, 0)),
        out_shape=jax.ShapeDtypeStruct((n, 50), F32),
    )(s64, pair, wt64, wpp, bpp, wpb, bp)


def _a_body(aa_ref, p0_ref, p1_ref, p2_ref, p3_ref,
            wt_ref, w0_ref, w1_ref, w2_ref, w3_ref, ba_ref, out_ref):
    acc = (jnp.dot(aa_ref[...], wt_ref[...], preferred_element_type=F32)
           + jnp.dot(p0_ref[...], w0_ref[...], preferred_element_type=F32)
           + jnp.dot(p1_ref[...], w1_ref[...], preferred_element_type=F32)
           + jnp.dot(p2_ref[...], w2_ref[...], preferred_element_type=F32)
           + jnp.dot(p3_ref[...], w3_ref[...], preferred_element_type=F32)
           + ba_ref[...])
    out_ref[...] = jax.nn.relu(acc)


def _tc_final_a(aa64, pasegs, wat, wabs, ba):
    n = aa64.shape[0]
    bm = 5000
    return pl.pallas_call(
        _a_body,
        grid=(n // bm,),
        in_specs=[
            pl.BlockSpec((bm, 64), lambda i: (i, 0)),
            pl.BlockSpec((bm, 16), lambda i: (i, 0)),
            pl.BlockSpec((bm, 16), lambda i: (i, 0)),
            pl.BlockSpec((bm, 16), lambda i: (i, 0)),
            pl.BlockSpec((bm, 16), lambda i: (i, 0)),
            pl.BlockSpec((64, 50), lambda i: (0, 0)),
            pl.BlockSpec((16, 50), lambda i: (0, 0)),
            pl.BlockSpec((16, 50), lambda i: (0, 0)),
            pl.BlockSpec((16, 50), lambda i: (0, 0)),
            pl.BlockSpec((16, 50), lambda i: (0, 0)),
            pl.BlockSpec((1, 50), lambda i: (0, 0)),
        ],
        out_specs=pl.BlockSpec((bm, 50), lambda i: (i, 0)),
        out_shape=jax.ShapeDtypeStruct((n, 50), F32),
    )(aa64, *pasegs, wat, *wabs, ba)


_SC_CHUNK = 64


def _s_gather_body(xcat_hbm, a2p_hbm, s_hbm, idx_v, rows_v, sbuf_v, sem):
    n_pairs = s_hbm.shape[0]
    total_steps = n_pairs // _SC_CHUNK
    wid = lax.axis_index("s") * NC + lax.axis_index("c")
    g_max = (total_steps + NW - 1) // NW

    def step_body(g, carry):
        step = wid + NW * g

        @pl.when(step < total_steps)
        def _():
            p0 = step * _SC_CHUNK
            pltpu.sync_copy(a2p_hbm.at[pl.ds(step * 2 * _SC_CHUNK, 2 * _SC_CHUNK)],
                            idx_v)
            pltpu.async_copy(xcat_hbm.at[idx_v], rows_v, sem).wait()

            def pair_body(p, c2):
                for k in range(4):
                    t_i = rows_v[2 * p, pl.ds(16 * k, 16)]
                    u_j = rows_v[2 * p + 1, pl.ds(64 + 16 * k, 16)]
                    t_j = rows_v[2 * p + 1, pl.ds(16 * k, 16)]
                    u_i = rows_v[2 * p, pl.ds(64 + 16 * k, 16)]
                    s = (jnp.maximum(t_i + u_j, 0.0)
                         + jnp.maximum(t_j + u_i, 0.0))
                    sbuf_v[p, pl.ds(16 * k, 16)] = s
                return c2

            lax.fori_loop(0, _SC_CHUNK, pair_body, 0)
            pltpu.sync_copy(sbuf_v, s_hbm.at[pl.ds(p0, _SC_CHUNK)])

        return carry

    lax.fori_loop(0, g_max, step_body, 0)


def _sc_gather_s(xcat, a2p_flat, n_pairs):
    mesh = plsc.VectorSubcoreMesh(core_axis_name="c", subcore_axis_name="s")
    return pl.kernel(
        _s_gather_body,
        out_type=jax.ShapeDtypeStruct((n_pairs, 64), F32),
        mesh=mesh,
        scratch_types=[
            pltpu.VMEM((2 * _SC_CHUNK,), jnp.int32),
            pltpu.VMEM((2 * _SC_CHUNK, 128), F32),
            pltpu.VMEM((_SC_CHUNK, 64), F32),
            pltpu.SemaphoreType.DMA,
        ],
    )(xcat, a2p_flat)


_SEG_CHUNK = 64
_ZROWS = 1000


def _segsum_body(pa0_hbm, pa1_hbm, pa2_hbm, pa3_hbm, keys_hbm,
                 out0_hbm, out1_hbm, out2_hbm, out3_hbm,
                 keys_v, rows_v, zbuf_v, wbuf_v, acc_sh):
    n_pairs = keys_hbm.shape[0]
    n_atoms = out0_hbm.shape[0]
    c = lax.axis_index("c")
    sid = lax.axis_index("s")

    def zero_buf(r, carry):
        zbuf_v[r, pl.ds(0, 16)] = jnp.zeros((16,), F32)
        return carry

    lax.fori_loop(0, _ZROWS, zero_buf, 0)
    n_chunks = n_atoms // _ZROWS
    total_steps = n_pairs // _SEG_CHUNK
    g_max = (total_steps + NS - 1) // NS

    for ph in range(2):
        for m in range((n_chunks + NS - 1) // NS):
            cz = sid + NS * m

            @pl.when(cz < n_chunks)
            def _():
                pltpu.sync_copy(zbuf_v, acc_sh.at[pl.ds(cz * _ZROWS, _ZROWS)])

        plsc.subcore_barrier()

        def step_body(g, carry):
            step = sid + NS * g

            @pl.when(step < total_steps)
            def _():
                base = step * _SEG_CHUNK
                pltpu.sync_copy(keys_hbm.at[pl.ds(base, _SEG_CHUNK)], keys_v)

                @pl.when(c == 0)
                def _():
                    pltpu.sync_copy(
                        (pa0_hbm if ph == 0 else pa2_hbm).at[
                            pl.ds(base, _SEG_CHUNK)], rows_v)

                @pl.when(c == 1)
                def _():
                    pltpu.sync_copy(
                        (pa1_hbm if ph == 0 else pa3_hbm).at[
                            pl.ds(base, _SEG_CHUNK)], rows_v)

                pltpu.sync_copy(rows_v, acc_sh.at[keys_v], add=True)

            return carry

        lax.fori_loop(0, g_max, step_body, 0)
        plsc.subcore_barrier()

        for m in range((n_chunks + NS - 1) // NS):
            cz = sid + NS * m

            @pl.when(cz < n_chunks)
            def _():
                r0 = cz * _ZROWS
                pltpu.sync_copy(acc_sh.at[pl.ds(r0, _ZROWS)], wbuf_v)

                @pl.when(c == 0)
                def _():
                    pltpu.sync_copy(
                        wbuf_v, (out0_hbm if ph == 0 else out2_hbm).at[
                            pl.ds(r0, _ZROWS)])

                @pl.when(c == 1)
                def _():
                    pltpu.sync_copy(
                        wbuf_v, (out1_hbm if ph == 0 else out3_hbm).at[
                            pl.ds(r0, _ZROWS)])

        plsc.subcore_barrier()


def _sc_segsum(pa0, pa1, pa2, pa3, pair_split, n_atoms):
    mesh = plsc.VectorSubcoreMesh(core_axis_name="c", subcore_axis_name="s")
    return pl.kernel(
        _segsum_body,
        out_type=[jax.ShapeDtypeStruct((n_atoms, 16), F32)] * 4,
        mesh=mesh,
        scratch_types=[
            pltpu.VMEM((_SEG_CHUNK,), jnp.int32),
            pltpu.VMEM((_SEG_CHUNK, 16), F32),
            pltpu.VMEM((_ZROWS, 16), F32),
            pltpu.VMEM((_ZROWS, 16), F32),
            pltpu.VMEM_SHARED((n_atoms, 16), F32),
        ],
        compiler_params=pltpu.CompilerParams(use_tc_tiling_on_sc=False),
    )(pa0, pa1, pa2, pa3, pair_split)


def kernel(atom_features, pair_features, pair_split, atom_to_pair,
           W_AA, b_AA, W_PA, b_PA, W_A, b_A,
           W_AP, b_AP, W_PP, b_PP, W_P, b_P):
    n_atoms, n_atom_in = atom_features.shape
    n_pairs = pair_features.shape[0]

    w1 = jnp.zeros((n_atom_in, 128), F32)
    w1 = w1.at[:, 0:50].set(W_AP[:n_atom_in])
    w1 = w1.at[:, 64:114].set(W_AP[n_atom_in:])
    b1 = jnp.zeros((1, 128), F32).at[0, 64:114].set(b_AP)
    w2 = jnp.zeros((n_atom_in, 64), F32).at[:, 0:50].set(W_AA)
    b2 = jnp.zeros((1, 64), F32).at[0, 0:50].set(b_AA)
    w3 = jnp.zeros((14, 64), F32).at[:, 0:50].set(W_PA)
    b3 = jnp.zeros((1, 64), F32).at[0, 0:50].set(b_PA)
    wt64 = jnp.zeros((64, 50), F32).at[0:50].set(W_P[:50])
    wpb = W_P[50:100]
    wat = jnp.zeros((64, 50), F32).at[0:50].set(W_A[:50])
    wabs = [W_A[50:66], W_A[66:82], W_A[82:98],
            jnp.zeros((16, 50), F32).at[0:2].set(W_A[98:100])]

    a2p_flat = atom_to_pair.reshape(-1).astype(jnp.int32)
    keys = pair_split.astype(jnp.int32)

    xcat, aa64 = _tc_prep(atom_features, w1, b1, w2, b2)
    pas = _tc_pair_dense(pair_features, w3, b3)

    s64 = _sc_gather_s(xcat, a2p_flat, n_pairs)
    pasegs = _sc_segsum(*pas, keys, n_atoms)

    p_out = _tc_final_p(s64, pair_features, wt64, W_PP, b_PP.reshape(1, 50),
                        wpb, b_P.reshape(1, 50))
    a_out = _tc_final_a(aa64, pasegs, wat, wabs, b_A.reshape(1, 50))
    return (a_out, p_out)


# speedup vs baseline: 3.9413x; 1.7957x over previous
import functools

import jax
import jax.numpy as jnp
from jax import lax
from jax.experimental import pallas as pl
from jax.experimental.pallas import tpu as pltpu
from jax.experimental.pallas import tpu_sc as plsc

F32 = jnp.float32

NC, NS, L = 2, 16, 16
NW = NC * NS


def _prep_body(atom_ref, w1_ref, b1_ref, w2_ref, b2_ref, xcat_ref, aa_ref):
    a = atom_ref[...]
    xcat_ref[...] = jnp.dot(a, w1_ref[...], preferred_element_type=F32) + b1_ref[...]
    aa_ref[...] = jax.nn.relu(
        jnp.dot(a, w2_ref[...], preferred_element_type=F32) + b2_ref[...])


def _tc_prep(atom, w1, b1, w2, b2):
    n, k = atom.shape
    bm = 2000
    grid = (n // bm,)
    return pl.pallas_call(
        _prep_body,
        grid=grid,
        in_specs=[
            pl.BlockSpec((bm, k), lambda i: (i, 0)),
            pl.BlockSpec((k, 128), lambda i: (0, 0)),
            pl.BlockSpec((1, 128), lambda i: (0, 0)),
            pl.BlockSpec((k, 64), lambda i: (0, 0)),
            pl.BlockSpec((1, 64), lambda i: (0, 0)),
        ],
        out_specs=[
            pl.BlockSpec((bm, 128), lambda i: (i, 0)),
            pl.BlockSpec((bm, 64), lambda i: (i, 0)),
        ],
        out_shape=[
            jax.ShapeDtypeStruct((n, 128), F32),
            jax.ShapeDtypeStruct((n, 64), F32),
        ],
    )(atom, w1, b1, w2, b2)


def _pa_body(pair_ref, w_ref, b_ref, out_ref):
    out_ref[...] = jax.nn.relu(
        jnp.dot(pair_ref[...], w_ref[...], preferred_element_type=F32) + b_ref[...])


def _tc_pair_dense(pair, w3, b3):
    n, k = pair.shape
    bm = 8000
    return pl.pallas_call(
        _pa_body,
        grid=(n // bm,),
        in_specs=[
            pl.BlockSpec((bm, k), lambda i: (i, 0)),
            pl.BlockSpec((k, 128), lambda i: (0, 0)),
            pl.BlockSpec((1, 128), lambda i: (0, 0)),
        ],
        out_specs=pl.BlockSpec((bm, 128), lambda i: (i, 0)),
        out_shape=jax.ShapeDtypeStruct((n, 128), F32),
    )(pair, w3, b3)


def _p_body(s_ref, pair_ref, wt_ref, wpp_ref, bpp_ref, wb_ref, bp_ref, out_ref):
    pp = jax.nn.relu(
        jnp.dot(pair_ref[...], wpp_ref[...], preferred_element_type=F32)
        + bpp_ref[...])
    acc = (jnp.dot(s_ref[...], wt_ref[...], preferred_element_type=F32)
           + jnp.dot(pp, wb_ref[...], preferred_element_type=F32) + bp_ref[...])
    out_ref[...] = jax.nn.relu(acc)


def _tc_final_p(s64, pair, wt64, wpp, bpp, wpb, bp):
    n = pair.shape[0]
    k = pair.shape[1]
    bm = 8000
    return pl.pallas_call(
        _p_body,
        grid=(n // bm,),
        in_specs=[
            pl.BlockSpec((bm, 64), lambda i: (i, 0)),
            pl.BlockSpec((bm, k), lambda i: (i, 0)),
            pl.BlockSpec((64, 50), lambda i: (0, 0)),
            pl.BlockSpec((k, 50), lambda i: (0, 0)),
            pl.BlockSpec((1, 50), lambda i: (0, 0)),
            pl.BlockSpec((50, 50), lambda i: (0, 0)),
            pl.BlockSpec((1, 50), lambda i: (0, 0)),
        ],
        out_specs=pl.BlockSpec((bm, 50), lambda i: (i, 0)),
        out_shape=jax.ShapeDtypeStruct((n, 50), F32),
    )(s64, pair, wt64, wpp, bpp, wpb, bp)


def _a_body(aa_ref, p0_ref, p1_ref, p2_ref, p3_ref,
            wt_ref, w0_ref, w1_ref, w2_ref, w3_ref, ba_ref, out_ref):
    acc = (jnp.dot(aa_ref[...], wt_ref[...], preferred_element_type=F32)
           + jnp.dot(p0_ref[...], w0_ref[...], preferred_element_type=F32)
           + jnp.dot(p1_ref[...], w1_ref[...], preferred_element_type=F32)
           + jnp.dot(p2_ref[...], w2_ref[...], preferred_element_type=F32)
           + jnp.dot(p3_ref[...], w3_ref[...], preferred_element_type=F32)
           + ba_ref[...])
    out_ref[...] = jax.nn.relu(acc)


def _tc_final_a(aa64, pasegs, wat, wabs, ba):
    n = aa64.shape[0]
    bm = 5000
    return pl.pallas_call(
        _a_body,
        grid=(n // bm,),
        in_specs=[
            pl.BlockSpec((bm, 64), lambda i: (i, 0)),
            pl.BlockSpec((bm, 16), lambda i: (i, 0)),
            pl.BlockSpec((bm, 16), lambda i: (i, 0)),
            pl.BlockSpec((bm, 16), lambda i: (i, 0)),
            pl.BlockSpec((bm, 16), lambda i: (i, 0)),
            pl.BlockSpec((64, 50), lambda i: (0, 0)),
            pl.BlockSpec((16, 50), lambda i: (0, 0)),
            pl.BlockSpec((16, 50), lambda i: (0, 0)),
            pl.BlockSpec((16, 50), lambda i: (0, 0)),
            pl.BlockSpec((16, 50), lambda i: (0, 0)),
            pl.BlockSpec((1, 50), lambda i: (0, 0)),
        ],
        out_specs=pl.BlockSpec((bm, 50), lambda i: (i, 0)),
        out_shape=jax.ShapeDtypeStruct((n, 50), F32),
    )(aa64, *pasegs, wat, *wabs, ba)


_SC_CHUNK = 64
_NB = 4


def _s_gather_body(xcat_hbm, a2p_hbm, s_hbm, idx_v, rows_v, sbuf_v, *sems):
    semi, semg, semo = sems[0:_NB], sems[_NB:2 * _NB], sems[2 * _NB:3 * _NB]
    n_pairs = s_hbm.shape[0]
    total_steps = n_pairs // _SC_CHUNK
    wid = lax.axis_index("s") * NC + lax.axis_index("c")
    ngroups = (total_steps + NW * _NB - 1) // (NW * _NB)

    def group_body(g2, carry):
        steps = [wid + NW * (_NB * g2 + b) for b in range(_NB)]

        for b in range(_NB):
            @pl.when(steps[b] < total_steps)
            def _(b=b):
                pltpu.async_copy(
                    a2p_hbm.at[pl.ds(steps[b] * 2 * _SC_CHUNK, 2 * _SC_CHUNK)],
                    idx_v.at[b], semi[b])

        for b in range(_NB):
            @pl.when(steps[b] < total_steps)
            def _(b=b):
                pltpu.make_async_copy(
                    a2p_hbm.at[pl.ds(steps[b] * 2 * _SC_CHUNK, 2 * _SC_CHUNK)],
                    idx_v.at[b], semi[b]).wait()
                pltpu.async_copy(xcat_hbm.at[idx_v.at[b]], rows_v.at[b], semg[b])

        for b in range(_NB):
            prev_ok = jnp.logical_and(g2 > 0,
                                      steps[b] - NW * _NB < total_steps)

            @pl.when(prev_ok)
            def _(b=b):
                pltpu.make_async_copy(
                    sbuf_v.at[b], s_hbm.at[pl.ds(0, _SC_CHUNK)],
                    semo[b]).wait()

            @pl.when(steps[b] < total_steps)
            def _(b=b):
                pltpu.make_async_copy(xcat_hbm.at[idx_v.at[b]], rows_v.at[b],
                                      semg[b]).wait()

                rb = rows_v.at[b]
                sb = sbuf_v.at[b]

                def pair_body(p, c2):
                    for k in range(4):
                        t_i = rb[2 * p, pl.ds(16 * k, 16)]
                        u_j = rb[2 * p + 1, pl.ds(64 + 16 * k, 16)]
                        t_j = rb[2 * p + 1, pl.ds(16 * k, 16)]
                        u_i = rb[2 * p, pl.ds(64 + 16 * k, 16)]
                        s = (jnp.maximum(t_i + u_j, 0.0)
                             + jnp.maximum(t_j + u_i, 0.0))
                        sb[p, pl.ds(16 * k, 16)] = s
                    return c2

                lax.fori_loop(0, _SC_CHUNK, pair_body, 0)
                pltpu.async_copy(
                    sbuf_v.at[b],
                    s_hbm.at[pl.ds(steps[b] * _SC_CHUNK, _SC_CHUNK)], semo[b])

        return carry

    lax.fori_loop(0, ngroups, group_body, 0)

    for b in range(_NB):
        last_step = wid + NW * (_NB * (ngroups - 1) + b)

        @pl.when(last_step < total_steps)
        def _(b=b):
            pltpu.make_async_copy(sbuf_v.at[b], s_hbm.at[pl.ds(0, _SC_CHUNK)],
                                  semo[b]).wait()


def _sc_gather_s(xcat, a2p_flat, n_pairs):
    mesh = plsc.VectorSubcoreMesh(core_axis_name="c", subcore_axis_name="s")
    return pl.kernel(
        _s_gather_body,
        out_type=jax.ShapeDtypeStruct((n_pairs, 64), F32),
        mesh=mesh,
        scratch_types=[
            pltpu.VMEM((_NB, 2 * _SC_CHUNK), jnp.int32),
            pltpu.VMEM((_NB, 2 * _SC_CHUNK, 128), F32),
            pltpu.VMEM((_NB, _SC_CHUNK, 64), F32),
        ] + [pltpu.SemaphoreType.DMA] * (3 * _NB),
    )(xcat, a2p_flat)


_SEG_CHUNK = 128
_ZROWS = 1000


def _segsum_body(pa_hbm, keys_hbm, out0_hbm, out1_hbm, out2_hbm, out3_hbm,
                 keys_v, rows_v, zbuf_v, wbuf_v, *sems):
    acc_sh = sems[-1]
    semk = sems[0:2]
    semr = sems[2:4]
    sems_ = sems[4:6]
    n_pairs = keys_hbm.shape[0]
    n_atoms = out0_hbm.shape[0]
    c = lax.axis_index("c")
    sid = lax.axis_index("s")

    def zero_buf(r, carry):
        zbuf_v[r, pl.ds(0, 16)] = jnp.zeros((16,), F32)
        return carry

    lax.fori_loop(0, _ZROWS, zero_buf, 0)
    n_chunks = n_atoms // _ZROWS
    total_steps = n_pairs // _SEG_CHUNK
    ngroups = (total_steps + NS * 2 - 1) // (NS * 2)

    for ph in range(2):
        col0 = 16 * (2 * ph) * 1 + 16 * c

        for m in range((n_chunks + NS - 1) // NS):
            cz = sid + NS * m

            @pl.when(cz < n_chunks)
            def _():
                pltpu.sync_copy(zbuf_v, acc_sh.at[pl.ds(cz * _ZROWS, _ZROWS)])

        plsc.subcore_barrier()

        def group_body(g2, carry):
            steps = [sid + NS * (2 * g2 + b) for b in range(2)]

            for b in range(2):
                prev_ok = jnp.logical_and(g2 > 0,
                                          steps[b] - NS * 2 < total_steps)

                @pl.when(prev_ok)
                def _(b=b):
                    pltpu.make_async_copy(
                        rows_v.at[b], acc_sh.at[keys_v.at[b]],
                        sems_[b]).wait()

                @pl.when(steps[b] < total_steps)
                def _(b=b):
                    base = steps[b] * _SEG_CHUNK
                    pltpu.async_copy(keys_hbm.at[pl.ds(base, _SEG_CHUNK)],
                                     keys_v.at[b], semk[b])
                    pltpu.async_copy(
                        pa_hbm.at[pl.ds(base, _SEG_CHUNK), pl.ds(col0, 16)],
                        rows_v.at[b], semr[b])

            for b in range(2):
                @pl.when(steps[b] < total_steps)
                def _(b=b):
                    base = steps[b] * _SEG_CHUNK
                    pltpu.make_async_copy(keys_hbm.at[pl.ds(base, _SEG_CHUNK)],
                                          keys_v.at[b], semk[b]).wait()
                    pltpu.make_async_copy(
                        pa_hbm.at[pl.ds(base, _SEG_CHUNK), pl.ds(col0, 16)],
                        rows_v.at[b], semr[b]).wait()
                    pltpu.async_copy(rows_v.at[b], acc_sh.at[keys_v.at[b]],
                                     sems_[b], add=True)

            return carry

        lax.fori_loop(0, ngroups, group_body, 0)

        for b in range(2):
            last_step = sid + NS * (2 * (ngroups - 1) + b)

            @pl.when(last_step < total_steps)
            def _(b=b):
                pltpu.make_async_copy(rows_v.at[b], acc_sh.at[keys_v.at[b]],
                                      sems_[b]).wait()

        plsc.subcore_barrier()

        for m in range((n_chunks + NS - 1) // NS):
            cz = sid + NS * m

            @pl.when(cz < n_chunks)
            def _():
                r0 = cz * _ZROWS
                pltpu.sync_copy(acc_sh.at[pl.ds(r0, _ZROWS)], wbuf_v)

                @pl.when(c == 0)
                def _():
                    pltpu.sync_copy(
                        wbuf_v, (out0_hbm if ph == 0 else out2_hbm).at[
                            pl.ds(r0, _ZROWS)])

                @pl.when(c == 1)
                def _():
                    pltpu.sync_copy(
                        wbuf_v, (out1_hbm if ph == 0 else out3_hbm).at[
                            pl.ds(r0, _ZROWS)])

        plsc.subcore_barrier()


def _sc_segsum(pa128, pair_split, n_atoms):
    mesh = plsc.VectorSubcoreMesh(core_axis_name="c", subcore_axis_name="s")
    return pl.kernel(
        _segsum_body,
        out_type=[jax.ShapeDtypeStruct((n_atoms, 16), F32)] * 4,
        mesh=mesh,
        scratch_types=[
            pltpu.VMEM((2, _SEG_CHUNK), jnp.int32),
            pltpu.VMEM((2, _SEG_CHUNK, 16), F32),
            pltpu.VMEM((_ZROWS, 16), F32),
            pltpu.VMEM((_ZROWS, 16), F32),
        ] + [pltpu.SemaphoreType.DMA] * 6 + [
            pltpu.VMEM_SHARED((n_atoms, 16), F32),
        ],
        compiler_params=pltpu.CompilerParams(use_tc_tiling_on_sc=False),
    )(pa128, pair_split)


def kernel(atom_features, pair_features, pair_split, atom_to_pair,
           W_AA, b_AA, W_PA, b_PA, W_A, b_A,
           W_AP, b_AP, W_PP, b_PP, W_P, b_P):
    n_atoms, n_atom_in = atom_features.shape
    n_pairs = pair_features.shape[0]

    w1 = jnp.zeros((n_atom_in, 128), F32)
    w1 = w1.at[:, 0:50].set(W_AP[:n_atom_in])
    w1 = w1.at[:, 64:114].set(W_AP[n_atom_in:])
    b1 = jnp.zeros((1, 128), F32).at[0, 64:114].set(b_AP)
    w2 = jnp.zeros((n_atom_in, 64), F32).at[:, 0:50].set(W_AA)
    b2 = jnp.zeros((1, 64), F32).at[0, 0:50].set(b_AA)
    w3 = jnp.zeros((14, 128), F32).at[:, 0:50].set(W_PA)
    b3 = jnp.zeros((1, 128), F32).at[0, 0:50].set(b_PA)
    wt64 = jnp.zeros((64, 50), F32).at[0:50].set(W_P[:50])
    wpb = W_P[50:100]
    wat = jnp.zeros((64, 50), F32).at[0:50].set(W_A[:50])
    wabs = [W_A[50:66], W_A[66:82], W_A[82:98],
            jnp.zeros((16, 50), F32).at[0:2].set(W_A[98:100])]

    a2p_flat = atom_to_pair.reshape(-1).astype(jnp.int32)
    keys = pair_split.astype(jnp.int32)

    xcat, aa64 = _tc_prep(atom_features, w1, b1, w2, b2)
    pa128 = _tc_pair_dense(pair_features, w3, b3)

    s64 = _sc_gather_s(xcat, a2p_flat, n_pairs)
    pasegs = _sc_segsum(pa128, keys, n_atoms)

    p_out = _tc_final_p(s64, pair_features, wt64, W_PP, b_PP.reshape(1, 50),
                        wpb, b_P.reshape(1, 50))
    a_out = _tc_final_a(aa64, pasegs, wat, wabs, b_A.reshape(1, 50))
    return (a_out, p_out)
